# baseline (speedup 1.0000x reference)
import functools

import jax
import jax.numpy as jnp
from jax import lax
from jax.experimental import pallas as pl
from jax.experimental.pallas import tpu as pltpu
from jax.experimental.pallas import tpu_sc as plsc

N = 10000
E = 320000
D = 128
G = 16
NC = 2
NS = 16
NW = NC * NS
EPW = E // NW
K = 80
CH = EPW // K
RPT = N // NS
BLK = 1000
NB = N // BLK
ZR = 25

_MESH = plsc.VectorSubcoreMesh(
    core_axis_name="c", subcore_axis_name="s", num_cores=NC, num_subcores=NS
)


@functools.partial(
    pl.kernel,
    out_type=jax.ShapeDtypeStruct((NB, NW, BLK), jnp.float32),
    mesh=_MESH,
    scratch_types=[
        pltpu.VMEM((EPW,), jnp.int32),
        pltpu.VMEM((EPW,), jnp.float32),
        pltpu.VMEM((NB, BLK), jnp.float32),
    ],
    compiler_params=pltpu.CompilerParams(
        needs_layout_passes=False, use_tc_tiling_on_sc=False),
)
def _sc_degree(row_hbm, w_hbm, z_hbm, out_hbm, idx_v, w_v, deg_v):
    c = lax.axis_index("c")
    s = lax.axis_index("s")
    wid = c * NS + s

    pltpu.sync_copy(z_hbm, deg_v)
    pltpu.sync_copy(row_hbm.at[wid], idx_v)
    pltpu.sync_copy(w_hbm.at[wid], w_v)

    def body(i, _):
        for u in range(5):
            idx = idx_v[pl.ds((5 * i + u) * 16, 16)]
            vals = jnp.abs(w_v[pl.ds((5 * i + u) * 16, 16)])
            plsc.addupdate_scatter(deg_v, [idx // BLK, idx % BLK], vals)
        return _

    lax.fori_loop(0, EPW // 80, body, None)
    for ib in range(NB):
        pltpu.sync_copy(deg_v.at[ib], out_hbm.at[ib, wid])


@functools.partial(
    pl.kernel,
    out_type=jax.ShapeDtypeStruct((NC, N, D), jnp.float32),
    mesh=_MESH,
    scratch_types=[
        pltpu.VMEM((CH, K), jnp.int32),
        pltpu.VMEM((CH, K), jnp.int32),
        pltpu.VMEM((K, D), jnp.float32),
        pltpu.VMEM((K, D), jnp.float32),
        pltpu.VMEM((ZR, D), jnp.float32),
        pltpu.VMEM_SHARED((N, D), jnp.float32),
        pltpu.SemaphoreType.DMA,
        pltpu.SemaphoreType.DMA,
    ],
    compiler_params=pltpu.CompilerParams(
        needs_layout_passes=False, use_tc_tiling_on_sc=False),
)
def _sc_propagate(g_hbm, row_hbm, col_hbm, out_hbm,
                  idxr_v, idxc_v, rows_a, rows_b, zrow_v, acc_sh,
                  sem_a, sem_b):
    c = lax.axis_index("c")
    s = lax.axis_index("s")

    zeros16 = jnp.zeros((16,), jnp.float32)

    def zero_body(i, _):
        for k in range(D // 16):
            zrow_v[i, pl.ds(k * 16, 16)] = zeros16
        return _

    lax.fori_loop(0, ZR, zero_body, None)

    pltpu.sync_copy(row_hbm.at[c, s], idxr_v)
    pltpu.sync_copy(col_hbm.at[c, s], idxc_v)

    def zfill(r, _):
        pltpu.sync_copy(zrow_v, acc_sh.at[pl.ds(s * RPT + r * ZR, ZR), :])
        return _

    lax.fori_loop(0, RPT // ZR, zfill, None)
    pltpu.async_copy(g_hbm.at[idxr_v.at[0]], rows_a, sem_a)
    plsc.subcore_barrier()

    def body(t, _):
        j0 = 2 * t
        pltpu.async_copy(g_hbm.at[idxr_v.at[j0 + 1]], rows_b, sem_b)
        pltpu.make_async_copy(g_hbm.at[idxr_v.at[j0]], rows_a, sem_a).wait()
        pltpu.sync_copy(rows_a, acc_sh.at[idxc_v.at[j0]], add=True)
        pltpu.async_copy(g_hbm.at[idxr_v.at[j0 + 2]], rows_a, sem_a)
        pltpu.make_async_copy(g_hbm.at[idxr_v.at[j0 + 1]], rows_b,
                              sem_b).wait()
        pltpu.sync_copy(rows_b, acc_sh.at[idxc_v.at[j0 + 1]], add=True)
        return _

    lax.fori_loop(0, (CH - 1) // 2, body, None)
    pltpu.make_async_copy(g_hbm.at[idxr_v.at[CH - 1]], rows_a, sem_a).wait()
    pltpu.sync_copy(rows_a, acc_sh.at[idxc_v.at[CH - 1]], add=True)
    plsc.subcore_barrier()
    pltpu.sync_copy(acc_sh.at[pl.ds(s * RPT, RPT), :],
                    out_hbm.at[c, pl.ds(s * RPT, RPT), :])


def _tc_mm_body(x_ref, w_ref, h_ref):
    h_ref[...] = lax.dot_general(x_ref[...], w_ref[...],
                                 (((1,), (1,)), ((), ())),
                                 preferred_element_type=jnp.float32)


def _tc_mm(x, w1):
    return pl.pallas_call(
        _tc_mm_body,
        grid=(N // BLK,),
        in_specs=[
            pl.BlockSpec((BLK, D), lambda i: (i, 0)),
            pl.BlockSpec((D, D), lambda i: (0, 0)),
        ],
        out_specs=pl.BlockSpec((BLK, D), lambda i: (i, 0)),
        out_shape=jax.ShapeDtypeStruct((N, D), jnp.float32),
    )(x, w1)


def _tc_scale_body(h_ref, deg_ref, g_ref, dinv_ref):
    deg = jnp.sum(deg_ref[0], axis=0)
    dinv = jnp.where(deg > 0, lax.rsqrt(jnp.where(deg > 0, deg, 1.0)), 0.0)
    db = jnp.broadcast_to(dinv[:, None], (BLK, D))
    g_ref[...] = h_ref[...] * db
    dinv_ref[...] = db


def _tc_scale(h, deg32):
    return pl.pallas_call(
        _tc_scale_body,
        grid=(N // BLK,),
        in_specs=[
            pl.BlockSpec((BLK, D), lambda i: (i, 0)),
            pl.BlockSpec((1, NW, BLK), lambda i: (i, 0, 0)),
        ],
        out_specs=[
            pl.BlockSpec((BLK, D), lambda i: (i, 0)),
            pl.BlockSpec((BLK, D), lambda i: (i, 0)),
        ],
        out_shape=[
            jax.ShapeDtypeStruct((N, D), jnp.float32),
            jax.ShapeDtypeStruct((N, D), jnp.float32),
        ],
    )(h, deg32)


def _tc_mid_body(p_ref, dinv_ref, b1_ref, w2_ref, g_ref):
    dinv = dinv_ref[...]
    t = dinv * (p_ref[0] + p_ref[1]) + b1_ref[...]
    t = jnp.where(t >= 0, t, 0.01 * t)
    h = lax.dot_general(t, w2_ref[...], (((1,), (1,)), ((), ())),
                        preferred_element_type=jnp.float32)
    g_ref[...] = h * dinv


def _tc_mid(p, dinvb, b1, w2):
    return pl.pallas_call(
        _tc_mid_body,
        grid=(N // BLK,),
        in_specs=[
            pl.BlockSpec((NC, BLK, D), lambda i: (0, i, 0)),
            pl.BlockSpec((BLK, D), lambda i: (i, 0)),
            pl.BlockSpec((1, D), lambda i: (0, 0)),
            pl.BlockSpec((D, D), lambda i: (0, 0)),
        ],
        out_specs=pl.BlockSpec((BLK, D), lambda i: (i, 0)),
        out_shape=jax.ShapeDtypeStruct((N, D), jnp.float32),
    )(p, dinvb, b1, w2)


def _tc_final_body(q_ref, dinv_ref, b2_ref, bat_ref, wm_ref, bm_ref,
                   out_ref, sums_ref, cnt_ref):
    i = pl.program_id(0)

    @pl.when(i == 0)
    def _():
        sums_ref[...] = jnp.zeros_like(sums_ref)
        cnt_ref[...] = jnp.zeros_like(cnt_ref)

    t = dinv_ref[...] * (q_ref[0] + q_ref[1]) + b2_ref[...]
    h = jnp.where(t >= 0, t, 0.01 * t)
    onehot = (lax.broadcasted_iota(jnp.int32, (BLK, G), 1)
              == bat_ref[...]).astype(jnp.float32)
    sums_ref[...] += lax.dot_general(onehot, h, (((0,), (0,)), ((), ())),
                                     preferred_element_type=jnp.float32)
    cnt_ref[...] += jnp.sum(onehot, axis=0)[:, None]

    @pl.when(i == N // BLK - 1)
    def _():
        pooled = sums_ref[...] / jnp.maximum(cnt_ref[...], 1.0)
        out_ref[...] = lax.dot_general(
            pooled, wm_ref[...], (((1,), (1,)), ((), ())),
            preferred_element_type=jnp.float32) + bm_ref[...]


def _tc_final(q, dinvb, b2, bat2d, wm, bm):
    return pl.pallas_call(
        _tc_final_body,
        grid=(N // BLK,),
        in_specs=[
            pl.BlockSpec((NC, BLK, D), lambda i: (0, i, 0)),
            pl.BlockSpec((BLK, D), lambda i: (i, 0)),
            pl.BlockSpec((1, D), lambda i: (0, 0)),
            pl.BlockSpec((BLK, 1), lambda i: (i, 0)),
            pl.BlockSpec((2, D), lambda i: (0, 0)),
            pl.BlockSpec((1, 2), lambda i: (0, 0)),
        ],
        out_specs=pl.BlockSpec((G, 2), lambda i: (0, 0)),
        out_shape=jax.ShapeDtypeStruct((G, 2), jnp.float32),
        scratch_shapes=[
            pltpu.VMEM((G, D), jnp.float32),
            pltpu.VMEM((G, 1), jnp.float32),
        ],
    )(q, dinvb, b2, bat2d, wm, bm)


def kernel(x, edge_index, edge_weights, batch, W1, b1, W2, b2, Wm, bm):
    row = edge_index[0].astype(jnp.int32)
    col = edge_index[1].astype(jnp.int32)
    row2 = row.reshape(NW, EPW)
    w2 = edge_weights.reshape(NW, EPW)
    row4 = row.reshape(NC, NS, CH, K)
    col4 = col.reshape(NC, NS, CH, K)

    deg32 = _sc_degree(row2, w2, jnp.zeros((NB, BLK), jnp.float32))
    h0 = _tc_mm(x, W1)
    g0, dinvb = _tc_scale(h0, deg32)
    p = _sc_propagate(g0, row4, col4)
    g1 = _tc_mid(p, dinvb, b1.reshape(1, D), W2)
    q = _sc_propagate(g1, row4, col4)
    return _tc_final(q, dinvb, b2.reshape(1, D),
                     batch.astype(jnp.int32).reshape(N, 1), Wm,
                     bm.reshape(1, 2))


# speedup vs baseline: 1.1775x; 1.1775x over previous
import functools

import jax
import jax.numpy as jnp
from jax import lax
from jax.experimental import pallas as pl
from jax.experimental.pallas import tpu as pltpu
from jax.experimental.pallas import tpu_sc as plsc

N = 10000
E = 320000
D = 128
G = 16
NC = 2
NS = 16
NW = NC * NS
EPW = E // NW
K = 40
PIPE = 5
CH = EPW // K
RPT = N // NS
BLK = 1000
NB = N // BLK
ZR = 25

_MESH = plsc.VectorSubcoreMesh(
    core_axis_name="c", subcore_axis_name="s", num_cores=NC, num_subcores=NS
)


@functools.partial(
    pl.kernel,
    out_type=jax.ShapeDtypeStruct((NB, NW, BLK), jnp.float32),
    mesh=_MESH,
    scratch_types=[
        pltpu.VMEM((EPW,), jnp.int32),
        pltpu.VMEM((EPW,), jnp.float32),
        pltpu.VMEM((NB, BLK), jnp.float32),
    ],
    compiler_params=pltpu.CompilerParams(
        needs_layout_passes=False, use_tc_tiling_on_sc=False),
)
def _sc_degree(row_hbm, w_hbm, z_hbm, out_hbm, idx_v, w_v, deg_v):
    c = lax.axis_index("c")
    s = lax.axis_index("s")
    wid = c * NS + s

    pltpu.sync_copy(z_hbm, deg_v)
    pltpu.sync_copy(row_hbm.at[wid], idx_v)
    pltpu.sync_copy(w_hbm.at[wid], w_v)

    def body(i, _):
        for u in range(5):
            idx = idx_v[pl.ds((5 * i + u) * 16, 16)]
            vals = jnp.abs(w_v[pl.ds((5 * i + u) * 16, 16)])
            plsc.addupdate_scatter(deg_v, [idx // BLK, idx % BLK], vals)
        return _

    lax.fori_loop(0, EPW // 80, body, None)
    for ib in range(NB):
        pltpu.sync_copy(deg_v.at[ib], out_hbm.at[ib, wid])


@functools.partial(
    pl.kernel,
    out_type=jax.ShapeDtypeStruct((NC, N, D), jnp.float32),
    mesh=_MESH,
    scratch_types=[
        pltpu.VMEM((CH, K), jnp.int32),
        pltpu.VMEM((CH, K), jnp.int32),
        [pltpu.VMEM((K, D), jnp.float32) for _ in range(PIPE)],
        pltpu.VMEM((ZR, D), jnp.float32),
        pltpu.VMEM_SHARED((N, D), jnp.float32),
        [pltpu.SemaphoreType.DMA for _ in range(PIPE)],
    ],
    compiler_params=pltpu.CompilerParams(
        needs_layout_passes=False, use_tc_tiling_on_sc=False),
)
def _sc_propagate(g_hbm, row_hbm, col_hbm, out_hbm,
                  idxr_v, idxc_v, rows, zrow_v, acc_sh, sems):
    c = lax.axis_index("c")
    s = lax.axis_index("s")

    zeros16 = jnp.zeros((16,), jnp.float32)

    def zero_body(i, _):
        for k in range(D // 16):
            zrow_v[i, pl.ds(k * 16, 16)] = zeros16
        return _

    lax.fori_loop(0, ZR, zero_body, None)

    pltpu.sync_copy(row_hbm.at[c, s], idxr_v)
    pltpu.sync_copy(col_hbm.at[c, s], idxc_v)

    def zfill(r, _):
        pltpu.sync_copy(zrow_v, acc_sh.at[pl.ds(s * RPT + r * ZR, ZR), :])
        return _

    lax.fori_loop(0, RPT // ZR, zfill, None)
    for u in range(PIPE):
        pltpu.async_copy(g_hbm.at[idxr_v.at[u]], rows[u], sems[u])
    plsc.subcore_barrier()

    def body(t, _):
        for u in range(PIPE):
            j = PIPE * t + u
            pltpu.make_async_copy(g_hbm.at[idxr_v.at[j]], rows[u],
                                  sems[u]).wait()
            pltpu.sync_copy(rows[u], acc_sh.at[idxc_v.at[j]], add=True)
            pltpu.async_copy(g_hbm.at[idxr_v.at[j + PIPE]], rows[u], sems[u])
        return _

    lax.fori_loop(0, CH // PIPE - 1, body, None)
    for u in range(PIPE):
        j = CH - PIPE + u
        pltpu.make_async_copy(g_hbm.at[idxr_v.at[j]], rows[u], sems[u]).wait()
        pltpu.sync_copy(rows[u], acc_sh.at[idxc_v.at[j]], add=True)
    plsc.subcore_barrier()
    pltpu.sync_copy(acc_sh.at[pl.ds(s * RPT, RPT), :],
                    out_hbm.at[c, pl.ds(s * RPT, RPT), :])


def _tc_mm_body(x_ref, w_ref, h_ref):
    h_ref[...] = lax.dot_general(x_ref[...], w_ref[...],
                                 (((1,), (1,)), ((), ())),
                                 preferred_element_type=jnp.float32)


def _tc_mm(x, w1):
    return pl.pallas_call(
        _tc_mm_body,
        grid=(N // BLK,),
        in_specs=[
            pl.BlockSpec((BLK, D), lambda i: (i, 0)),
            pl.BlockSpec((D, D), lambda i: (0, 0)),
        ],
        out_specs=pl.BlockSpec((BLK, D), lambda i: (i, 0)),
        out_shape=jax.ShapeDtypeStruct((N, D), jnp.float32),
    )(x, w1)


def _dinv_block(deg_ref):
    deg = jnp.sum(deg_ref[0], axis=0)
    return jnp.where(deg > 0, lax.rsqrt(jnp.where(deg > 0, deg, 1.0)), 0.0)


def _tc_scale_body(h_ref, deg_ref, g_ref):
    g_ref[...] = h_ref[...] * _dinv_block(deg_ref)[:, None]


def _tc_scale(h, deg32):
    return pl.pallas_call(
        _tc_scale_body,
        grid=(N // BLK,),
        in_specs=[
            pl.BlockSpec((BLK, D), lambda i: (i, 0)),
            pl.BlockSpec((1, NW, BLK), lambda i: (i, 0, 0)),
        ],
        out_specs=pl.BlockSpec((BLK, D), lambda i: (i, 0)),
        out_shape=jax.ShapeDtypeStruct((N, D), jnp.float32),
    )(h, deg32)


def _tc_mid_body(p_ref, deg_ref, b1_ref, w2_ref, g_ref):
    dinv = _dinv_block(deg_ref)[:, None]
    t = dinv * (p_ref[0] + p_ref[1]) + b1_ref[...]
    t = jnp.where(t >= 0, t, 0.01 * t)
    h = lax.dot_general(t, w2_ref[...], (((1,), (1,)), ((), ())),
                        preferred_element_type=jnp.float32)
    g_ref[...] = h * dinv


def _tc_mid(p, deg32, b1, w2):
    return pl.pallas_call(
        _tc_mid_body,
        grid=(N // BLK,),
        in_specs=[
            pl.BlockSpec((NC, BLK, D), lambda i: (0, i, 0)),
            pl.BlockSpec((1, NW, BLK), lambda i: (i, 0, 0)),
            pl.BlockSpec((1, D), lambda i: (0, 0)),
            pl.BlockSpec((D, D), lambda i: (0, 0)),
        ],
        out_specs=pl.BlockSpec((BLK, D), lambda i: (i, 0)),
        out_shape=jax.ShapeDtypeStruct((N, D), jnp.float32),
    )(p, deg32, b1, w2)


def _tc_final_body(q_ref, deg_ref, b2_ref, bat_ref, wm_ref, bm_ref,
                   out_ref, sums_ref, cnt_ref):
    i = pl.program_id(0)

    @pl.when(i == 0)
    def _():
        sums_ref[...] = jnp.zeros_like(sums_ref)
        cnt_ref[...] = jnp.zeros_like(cnt_ref)

    t = _dinv_block(deg_ref)[:, None] * (q_ref[0] + q_ref[1]) + b2_ref[...]
    h = jnp.where(t >= 0, t, 0.01 * t)
    onehot = (lax.broadcasted_iota(jnp.int32, (BLK, G), 1)
              == bat_ref[...]).astype(jnp.float32)
    sums_ref[...] += lax.dot_general(onehot, h, (((0,), (0,)), ((), ())),
                                     preferred_element_type=jnp.float32)
    cnt_ref[...] += jnp.sum(onehot, axis=0)[:, None]

    @pl.when(i == N // BLK - 1)
    def _():
        pooled = sums_ref[...] / jnp.maximum(cnt_ref[...], 1.0)
        out_ref[...] = lax.dot_general(
            pooled, wm_ref[...], (((1,), (1,)), ((), ())),
            preferred_element_type=jnp.float32) + bm_ref[...]


def _tc_final(q, deg32, b2, bat2d, wm, bm):
    return pl.pallas_call(
        _tc_final_body,
        grid=(N // BLK,),
        in_specs=[
            pl.BlockSpec((NC, BLK, D), lambda i: (0, i, 0)),
            pl.BlockSpec((1, NW, BLK), lambda i: (i, 0, 0)),
            pl.BlockSpec((1, D), lambda i: (0, 0)),
            pl.BlockSpec((BLK, 1), lambda i: (i, 0)),
            pl.BlockSpec((2, D), lambda i: (0, 0)),
            pl.BlockSpec((1, 2), lambda i: (0, 0)),
        ],
        out_specs=pl.BlockSpec((G, 2), lambda i: (0, 0)),
        out_shape=jax.ShapeDtypeStruct((G, 2), jnp.float32),
        scratch_shapes=[
            pltpu.VMEM((G, D), jnp.float32),
            pltpu.VMEM((G, 1), jnp.float32),
        ],
    )(q, deg32, b2, bat2d, wm, bm)


def kernel(x, edge_index, edge_weights, batch, W1, b1, W2, b2, Wm, bm):
    row = edge_index[0].astype(jnp.int32)
    col = edge_index[1].astype(jnp.int32)
    row2 = row.reshape(NW, EPW)
    w2 = edge_weights.reshape(NW, EPW)
    row4 = row.reshape(NC, NS, CH, K)
    col4 = col.reshape(NC, NS, CH, K)

    deg32 = _sc_degree(row2, w2, jnp.zeros((NB, BLK), jnp.float32))
    h0 = _tc_mm(x, W1)
    g0 = _tc_scale(h0, deg32)
    p = _sc_propagate(g0, row4, col4)
    g1 = _tc_mid(p, deg32, b1.reshape(1, D), W2)
    q = _sc_propagate(g1, row4, col4)
    return _tc_final(q, deg32, b2.reshape(1, D),
                     batch.astype(jnp.int32).reshape(N, 1), Wm,
                     bm.reshape(1, 2))


# speedup vs baseline: 1.2781x; 1.0855x over previous
import functools

import jax
import jax.numpy as jnp
from jax import lax
from jax.experimental import pallas as pl
from jax.experimental.pallas import tpu as pltpu
from jax.experimental.pallas import tpu_sc as plsc

N = 10000
E = 320000
D = 128
G = 16
NC = 2
NS = 16
NW = NC * NS
EPW = E // NW
K = 40
PIPE = 5
CH = EPW // K
RPT = N // NS
BLK = 1000
NB = N // BLK
ZR = 25

_MESH = plsc.VectorSubcoreMesh(
    core_axis_name="c", subcore_axis_name="s", num_cores=NC, num_subcores=NS
)


@functools.partial(
    pl.kernel,
    out_type=jax.ShapeDtypeStruct((NB, NW, BLK), jnp.float32),
    mesh=_MESH,
    scratch_types=[
        pltpu.VMEM((EPW,), jnp.int32),
        pltpu.VMEM((EPW,), jnp.float32),
        pltpu.VMEM((NB, BLK), jnp.float32),
    ],
    compiler_params=pltpu.CompilerParams(
        needs_layout_passes=False, use_tc_tiling_on_sc=False),
)
def _sc_degree(row_hbm, w_hbm, z_hbm, out_hbm, idx_v, w_v, deg_v):
    c = lax.axis_index("c")
    s = lax.axis_index("s")
    wid = c * NS + s

    pltpu.sync_copy(z_hbm, deg_v)
    pltpu.sync_copy(row_hbm.at[wid], idx_v)
    pltpu.sync_copy(w_hbm.at[wid], w_v)

    def body(i, _):
        for u in range(5):
            idx = idx_v[pl.ds((5 * i + u) * 16, 16)]
            vals = jnp.abs(w_v[pl.ds((5 * i + u) * 16, 16)])
            ib = lax.shift_right_logical(idx * 8389, 23)
            plsc.addupdate_scatter(deg_v, [ib, idx - ib * BLK], vals)
        return _

    lax.fori_loop(0, EPW // 80, body, None)
    for ib in range(NB):
        pltpu.sync_copy(deg_v.at[ib], out_hbm.at[ib, wid])


@functools.partial(
    pl.kernel,
    out_type=jax.ShapeDtypeStruct((NC, N, D), jnp.float32),
    mesh=_MESH,
    scratch_types=[
        pltpu.VMEM((CH, K), jnp.int32),
        pltpu.VMEM((CH, K), jnp.int32),
        [pltpu.VMEM((K, D), jnp.float32) for _ in range(PIPE)],
        pltpu.VMEM((ZR, D), jnp.float32),
        pltpu.VMEM_SHARED((N, D), jnp.float32),
        [pltpu.SemaphoreType.DMA for _ in range(PIPE)],
    ],
    compiler_params=pltpu.CompilerParams(
        needs_layout_passes=False, use_tc_tiling_on_sc=False),
)
def _sc_propagate(g_hbm, row_hbm, col_hbm, out_hbm,
                  idxr_v, idxc_v, rows, zrow_v, acc_sh, sems):
    c = lax.axis_index("c")
    s = lax.axis_index("s")

    zeros16 = jnp.zeros((16,), jnp.float32)

    def zero_body(i, _):
        for k in range(D // 16):
            zrow_v[i, pl.ds(k * 16, 16)] = zeros16
        return _

    lax.fori_loop(0, ZR, zero_body, None)

    pltpu.sync_copy(row_hbm.at[c, s], idxr_v)
    pltpu.sync_copy(col_hbm.at[c, s], idxc_v)

    def zfill(r, _):
        pltpu.sync_copy(zrow_v, acc_sh.at[pl.ds(s * RPT + r * ZR, ZR), :])
        return _

    lax.fori_loop(0, RPT // ZR, zfill, None)
    for u in range(PIPE):
        pltpu.async_copy(g_hbm.at[idxr_v.at[u]], rows[u], sems[u])
    plsc.subcore_barrier()

    def body(t, _):
        for u in range(PIPE):
            j = PIPE * t + u
            pltpu.make_async_copy(g_hbm.at[idxr_v.at[j]], rows[u],
                                  sems[u]).wait()
            pltpu.sync_copy(rows[u], acc_sh.at[idxc_v.at[j]], add=True)
            pltpu.async_copy(g_hbm.at[idxr_v.at[j + PIPE]], rows[u], sems[u])
        return _

    lax.fori_loop(0, CH // PIPE - 1, body, None)
    for u in range(PIPE):
        j = CH - PIPE + u
        pltpu.make_async_copy(g_hbm.at[idxr_v.at[j]], rows[u], sems[u]).wait()
        pltpu.sync_copy(rows[u], acc_sh.at[idxc_v.at[j]], add=True)
    plsc.subcore_barrier()
    pltpu.sync_copy(acc_sh.at[pl.ds(s * RPT, RPT), :],
                    out_hbm.at[c, pl.ds(s * RPT, RPT), :])


def _tc_mm_body(x_ref, w_ref, h_ref):
    h_ref[...] = lax.dot_general(x_ref[...], w_ref[...],
                                 (((1,), (1,)), ((), ())),
                                 preferred_element_type=jnp.float32)


def _tc_mm(x, w1):
    return pl.pallas_call(
        _tc_mm_body,
        grid=(N // BLK,),
        in_specs=[
            pl.BlockSpec((BLK, D), lambda i: (i, 0)),
            pl.BlockSpec((D, D), lambda i: (0, 0)),
        ],
        out_specs=pl.BlockSpec((BLK, D), lambda i: (i, 0)),
        out_shape=jax.ShapeDtypeStruct((N, D), jnp.float32),
    )(x, w1)


def _dinv_block(deg_ref):
    deg = jnp.sum(deg_ref[0], axis=0)
    return jnp.where(deg > 0, lax.rsqrt(jnp.where(deg > 0, deg, 1.0)), 0.0)


def _tc_scale_body(h_ref, deg_ref, g_ref):
    g_ref[...] = h_ref[...] * _dinv_block(deg_ref)[:, None]


def _tc_scale(h, deg32):
    return pl.pallas_call(
        _tc_scale_body,
        grid=(N // BLK,),
        in_specs=[
            pl.BlockSpec((BLK, D), lambda i: (i, 0)),
            pl.BlockSpec((1, NW, BLK), lambda i: (i, 0, 0)),
        ],
        out_specs=pl.BlockSpec((BLK, D), lambda i: (i, 0)),
        out_shape=jax.ShapeDtypeStruct((N, D), jnp.float32),
    )(h, deg32)


def _tc_mid_body(p_ref, deg_ref, b1_ref, w2_ref, g_ref):
    dinv = _dinv_block(deg_ref)[:, None]
    t = dinv * (p_ref[0] + p_ref[1]) + b1_ref[...]
    t = jnp.where(t >= 0, t, 0.01 * t)
    h = lax.dot_general(t, w2_ref[...], (((1,), (1,)), ((), ())),
                        preferred_element_type=jnp.float32)
    g_ref[...] = h * dinv


def _tc_mid(p, deg32, b1, w2):
    return pl.pallas_call(
        _tc_mid_body,
        grid=(N // BLK,),
        in_specs=[
            pl.BlockSpec((NC, BLK, D), lambda i: (0, i, 0)),
            pl.BlockSpec((1, NW, BLK), lambda i: (i, 0, 0)),
            pl.BlockSpec((1, D), lambda i: (0, 0)),
            pl.BlockSpec((D, D), lambda i: (0, 0)),
        ],
        out_specs=pl.BlockSpec((BLK, D), lambda i: (i, 0)),
        out_shape=jax.ShapeDtypeStruct((N, D), jnp.float32),
    )(p, deg32, b1, w2)


def _tc_final_body(q_ref, deg_ref, b2_ref, bat_ref, wm_ref, bm_ref,
                   out_ref, sums_ref, cnt_ref):
    i = pl.program_id(0)

    @pl.when(i == 0)
    def _():
        sums_ref[...] = jnp.zeros_like(sums_ref)
        cnt_ref[...] = jnp.zeros_like(cnt_ref)

    t = _dinv_block(deg_ref)[:, None] * (q_ref[0] + q_ref[1]) + b2_ref[...]
    h = jnp.where(t >= 0, t, 0.01 * t)
    onehot = (lax.broadcasted_iota(jnp.int32, (BLK, G), 1)
              == bat_ref[...]).astype(jnp.float32)
    sums_ref[...] += lax.dot_general(onehot, h, (((0,), (0,)), ((), ())),
                                     preferred_element_type=jnp.float32)
    cnt_ref[...] += jnp.sum(onehot, axis=0)[:, None]

    @pl.when(i == N // BLK - 1)
    def _():
        pooled = sums_ref[...] / jnp.maximum(cnt_ref[...], 1.0)
        out_ref[...] = lax.dot_general(
            pooled, wm_ref[...], (((1,), (1,)), ((), ())),
            preferred_element_type=jnp.float32) + bm_ref[...]


def _tc_final(q, deg32, b2, bat2d, wm, bm):
    return pl.pallas_call(
        _tc_final_body,
        grid=(N // BLK,),
        in_specs=[
            pl.BlockSpec((NC, BLK, D), lambda i: (0, i, 0)),
            pl.BlockSpec((1, NW, BLK), lambda i: (i, 0, 0)),
            pl.BlockSpec((1, D), lambda i: (0, 0)),
            pl.BlockSpec((BLK, 1), lambda i: (i, 0)),
            pl.BlockSpec((2, D), lambda i: (0, 0)),
            pl.BlockSpec((1, 2), lambda i: (0, 0)),
        ],
        out_specs=pl.BlockSpec((G, 2), lambda i: (0, 0)),
        out_shape=jax.ShapeDtypeStruct((G, 2), jnp.float32),
        scratch_shapes=[
            pltpu.VMEM((G, D), jnp.float32),
            pltpu.VMEM((G, 1), jnp.float32),
        ],
    )(q, deg32, b2, bat2d, wm, bm)


def kernel(x, edge_index, edge_weights, batch, W1, b1, W2, b2, Wm, bm):
    row = edge_index[0].astype(jnp.int32)
    col = edge_index[1].astype(jnp.int32)
    row2 = row.reshape(NW, EPW)
    w2 = edge_weights.reshape(NW, EPW)
    row4 = row.reshape(NC, NS, CH, K)
    col4 = col.reshape(NC, NS, CH, K)

    deg32 = _sc_degree(row2, w2, jnp.zeros((NB, BLK), jnp.float32))
    h0 = _tc_mm(x, W1)
    g0 = _tc_scale(h0, deg32)
    p = _sc_propagate(g0, row4, col4)
    g1 = _tc_mid(p, deg32, b1.reshape(1, D), W2)
    q = _sc_propagate(g1, row4, col4)
    return _tc_final(q, deg32, b2.reshape(1, D),
                     batch.astype(jnp.int32).reshape(N, 1), Wm,
                     bm.reshape(1, 2))


# speedup vs baseline: 1.3332x; 1.0431x over previous
import functools

import jax
import jax.numpy as jnp
from jax import lax
from jax.experimental import pallas as pl
from jax.experimental.pallas import tpu as pltpu
from jax.experimental.pallas import tpu_sc as plsc

N = 10000
E = 320000
D = 128
G = 16
NC = 2
NS = 16
NW = NC * NS
EPW = E // NW
K = 40
PIPE = 5
CH = EPW // K
RPT = N // NS
BLK = 1000
NB = N // BLK
ZR = 25

_MESH = plsc.VectorSubcoreMesh(
    core_axis_name="c", subcore_axis_name="s", num_cores=NC, num_subcores=NS
)


@functools.partial(
    pl.kernel,
    out_type=jax.ShapeDtypeStruct((NB, NW, BLK), jnp.float32),
    mesh=_MESH,
    scratch_types=[
        pltpu.VMEM((EPW,), jnp.int32),
        pltpu.VMEM((EPW,), jnp.float32),
        pltpu.VMEM((NB, BLK), jnp.float32),
    ],
    compiler_params=pltpu.CompilerParams(
        needs_layout_passes=False, use_tc_tiling_on_sc=False),
)
def _sc_degree(e2_hbm, w_hbm, z_hbm, out_hbm, idx_v, w_v, deg_v):
    c = lax.axis_index("c")
    s = lax.axis_index("s")
    wid = c * NS + s

    pltpu.sync_copy(z_hbm, deg_v)
    pltpu.sync_copy(e2_hbm.at[0, wid], idx_v)
    pltpu.sync_copy(w_hbm.at[wid], w_v)

    def body(i, _):
        for u in range(5):
            idx = idx_v[pl.ds((5 * i + u) * 16, 16)]
            vals = jnp.abs(w_v[pl.ds((5 * i + u) * 16, 16)])
            ib = lax.shift_right_logical(idx * 8389, 23)
            plsc.addupdate_scatter(deg_v, [ib, idx - ib * BLK], vals)
        return _

    lax.fori_loop(0, EPW // 80, body, None)
    for ib in range(NB):
        pltpu.sync_copy(deg_v.at[ib], out_hbm.at[ib, wid])


@functools.partial(
    pl.kernel,
    out_type=jax.ShapeDtypeStruct((NC, N, D), jnp.float32),
    mesh=_MESH,
    scratch_types=[
        pltpu.VMEM((CH, K), jnp.int32),
        pltpu.VMEM((CH, K), jnp.int32),
        [pltpu.VMEM((K, D), jnp.float32) for _ in range(PIPE)],
        pltpu.VMEM((ZR, D), jnp.float32),
        pltpu.VMEM_SHARED((N, D), jnp.float32),
        [pltpu.SemaphoreType.DMA for _ in range(PIPE)],
    ],
    compiler_params=pltpu.CompilerParams(
        needs_layout_passes=False, use_tc_tiling_on_sc=False),
)
def _sc_propagate(g_hbm, e4_hbm, out_hbm,
                  idxr_v, idxc_v, rows, zrow_v, acc_sh, sems):
    c = lax.axis_index("c")
    s = lax.axis_index("s")

    zeros16 = jnp.zeros((16,), jnp.float32)

    def zero_body(i, _):
        for k in range(D // 16):
            zrow_v[i, pl.ds(k * 16, 16)] = zeros16
        return _

    lax.fori_loop(0, ZR, zero_body, None)

    pltpu.sync_copy(e4_hbm.at[0, c, s], idxr_v)
    pltpu.sync_copy(e4_hbm.at[1, c, s], idxc_v)

    def zfill(r, _):
        pltpu.sync_copy(zrow_v, acc_sh.at[pl.ds(s * RPT + r * ZR, ZR), :])
        return _

    lax.fori_loop(0, RPT // ZR, zfill, None)
    for u in range(PIPE):
        pltpu.async_copy(g_hbm.at[idxr_v.at[u]], rows[u], sems[u])
    plsc.subcore_barrier()

    def body(t, _):
        for u in range(PIPE):
            j = PIPE * t + u
            pltpu.make_async_copy(g_hbm.at[idxr_v.at[j]], rows[u],
                                  sems[u]).wait()
            pltpu.sync_copy(rows[u], acc_sh.at[idxc_v.at[j]], add=True)
            pltpu.async_copy(g_hbm.at[idxr_v.at[j + PIPE]], rows[u], sems[u])
        return _

    lax.fori_loop(0, CH // PIPE - 1, body, None)
    for u in range(PIPE):
        j = CH - PIPE + u
        pltpu.make_async_copy(g_hbm.at[idxr_v.at[j]], rows[u], sems[u]).wait()
        pltpu.sync_copy(rows[u], acc_sh.at[idxc_v.at[j]], add=True)
    plsc.subcore_barrier()
    pltpu.sync_copy(acc_sh.at[pl.ds(s * RPT, RPT), :],
                    out_hbm.at[c, pl.ds(s * RPT, RPT), :])


def _tc_mm_body(x_ref, w_ref, h_ref):
    h_ref[...] = lax.dot_general(x_ref[...], w_ref[...],
                                 (((1,), (1,)), ((), ())),
                                 preferred_element_type=jnp.float32)


def _tc_mm(x, w1):
    return pl.pallas_call(
        _tc_mm_body,
        grid=(N // BLK,),
        in_specs=[
            pl.BlockSpec((BLK, D), lambda i: (i, 0)),
            pl.BlockSpec((D, D), lambda i: (0, 0)),
        ],
        out_specs=pl.BlockSpec((BLK, D), lambda i: (i, 0)),
        out_shape=jax.ShapeDtypeStruct((N, D), jnp.float32),
    )(x, w1)


def _dinv_block(deg_ref):
    deg = jnp.sum(deg_ref[0], axis=0)
    return jnp.where(deg > 0, lax.rsqrt(jnp.where(deg > 0, deg, 1.0)), 0.0)


def _tc_scale_body(h_ref, deg_ref, g_ref):
    g_ref[...] = h_ref[...] * _dinv_block(deg_ref)[:, None]


def _tc_scale(h, deg32):
    return pl.pallas_call(
        _tc_scale_body,
        grid=(N // BLK,),
        in_specs=[
            pl.BlockSpec((BLK, D), lambda i: (i, 0)),
            pl.BlockSpec((1, NW, BLK), lambda i: (i, 0, 0)),
        ],
        out_specs=pl.BlockSpec((BLK, D), lambda i: (i, 0)),
        out_shape=jax.ShapeDtypeStruct((N, D), jnp.float32),
    )(h, deg32)


def _tc_mid_body(p_ref, deg_ref, b1_ref, w2_ref, g_ref):
    dinv = _dinv_block(deg_ref)[:, None]
    t = dinv * (p_ref[0] + p_ref[1]) + b1_ref[...]
    t = jnp.where(t >= 0, t, 0.01 * t)
    h = lax.dot_general(t, w2_ref[...], (((1,), (1,)), ((), ())),
                        preferred_element_type=jnp.float32)
    g_ref[...] = h * dinv


def _tc_mid(p, deg32, b1, w2):
    return pl.pallas_call(
        _tc_mid_body,
        grid=(N // BLK,),
        in_specs=[
            pl.BlockSpec((NC, BLK, D), lambda i: (0, i, 0)),
            pl.BlockSpec((1, NW, BLK), lambda i: (i, 0, 0)),
            pl.BlockSpec((1, D), lambda i: (0, 0)),
            pl.BlockSpec((D, D), lambda i: (0, 0)),
        ],
        out_specs=pl.BlockSpec((BLK, D), lambda i: (i, 0)),
        out_shape=jax.ShapeDtypeStruct((N, D), jnp.float32),
    )(p, deg32, b1, w2)


def _tc_final_body(q_ref, deg_ref, b2_ref, bat_ref, wm_ref, bm_ref,
                   out_ref, sums_ref, cnt_ref):
    i = pl.program_id(0)

    @pl.when(i == 0)
    def _():
        sums_ref[...] = jnp.zeros_like(sums_ref)
        cnt_ref[...] = jnp.zeros_like(cnt_ref)

    t = _dinv_block(deg_ref)[:, None] * (q_ref[0] + q_ref[1]) + b2_ref[...]
    h = jnp.where(t >= 0, t, 0.01 * t)
    onehot = (lax.broadcasted_iota(jnp.int32, (BLK, G), 1)
              == bat_ref[...]).astype(jnp.float32)
    sums_ref[...] += lax.dot_general(onehot, h, (((0,), (0,)), ((), ())),
                                     preferred_element_type=jnp.float32)
    cnt_ref[...] += jnp.sum(onehot, axis=0)[:, None]

    @pl.when(i == N // BLK - 1)
    def _():
        pooled = sums_ref[...] / jnp.maximum(cnt_ref[...], 1.0)
        out_ref[...] = lax.dot_general(
            pooled, wm_ref[...], (((1,), (1,)), ((), ())),
            preferred_element_type=jnp.float32) + bm_ref[...]


def _tc_final(q, deg32, b2, bat2d, wm, bm):
    return pl.pallas_call(
        _tc_final_body,
        grid=(N // BLK,),
        in_specs=[
            pl.BlockSpec((NC, BLK, D), lambda i: (0, i, 0)),
            pl.BlockSpec((1, NW, BLK), lambda i: (i, 0, 0)),
            pl.BlockSpec((1, D), lambda i: (0, 0)),
            pl.BlockSpec((BLK, 1), lambda i: (i, 0)),
            pl.BlockSpec((2, D), lambda i: (0, 0)),
            pl.BlockSpec((1, 2), lambda i: (0, 0)),
        ],
        out_specs=pl.BlockSpec((G, 2), lambda i: (0, 0)),
        out_shape=jax.ShapeDtypeStruct((G, 2), jnp.float32),
        scratch_shapes=[
            pltpu.VMEM((G, D), jnp.float32),
            pltpu.VMEM((G, 1), jnp.float32),
        ],
    )(q, deg32, b2, bat2d, wm, bm)


def kernel(x, edge_index, edge_weights, batch, W1, b1, W2, b2, Wm, bm):
    ei = edge_index.astype(jnp.int32)
    e2 = ei.reshape(2, NW, EPW)
    e4 = ei.reshape(2, NC, NS, CH, K)
    w2 = edge_weights.reshape(NW, EPW)

    deg32 = _sc_degree(e2, w2, jnp.zeros((NB, BLK), jnp.float32))
    h0 = _tc_mm(x, W1)
    g0 = _tc_scale(h0, deg32)
    p = _sc_propagate(g0, e4)
    g1 = _tc_mid(p, deg32, b1.reshape(1, D), W2)
    q = _sc_propagate(g1, e4)
    return _tc_final(q, deg32, b2.reshape(1, D),
                     batch.astype(jnp.int32).reshape(N, 1), Wm,
                     bm.reshape(1, 2))


# speedup vs baseline: 1.3383x; 1.0038x over previous
import functools

import jax
import jax.numpy as jnp
from jax import lax
from jax.experimental import pallas as pl
from jax.experimental.pallas import tpu as pltpu
from jax.experimental.pallas import tpu_sc as plsc

N = 10000
E = 320000
D = 128
G = 16
NC = 2
NS = 16
NW = NC * NS
EPW = E // NW
K = 40
PIPE = 5
CH = EPW // K
RPT = N // NS
BLK = 1000
NB = N // BLK
ZR = 25

_MESH = plsc.VectorSubcoreMesh(
    core_axis_name="c", subcore_axis_name="s", num_cores=NC, num_subcores=NS
)


@functools.partial(
    pl.kernel,
    out_type=jax.ShapeDtypeStruct((NB, NW, BLK), jnp.float32),
    mesh=_MESH,
    scratch_types=[
        pltpu.VMEM((EPW,), jnp.int32),
        pltpu.VMEM((EPW,), jnp.float32),
        pltpu.VMEM((NB, BLK), jnp.float32),
    ],
    compiler_params=pltpu.CompilerParams(
        needs_layout_passes=False, use_tc_tiling_on_sc=False),
)
def _sc_degree(e2_hbm, w_hbm, z_hbm, out_hbm, idx_v, w_v, deg_v):
    c = lax.axis_index("c")
    s = lax.axis_index("s")
    wid = c * NS + s

    pltpu.sync_copy(z_hbm, deg_v)
    pltpu.sync_copy(e2_hbm.at[0, wid], idx_v)
    pltpu.sync_copy(w_hbm.at[wid], w_v)

    def body(i, _):
        for u in range(5):
            idx = idx_v[pl.ds((5 * i + u) * 16, 16)]
            vals = jnp.abs(w_v[pl.ds((5 * i + u) * 16, 16)])
            ib = lax.shift_right_logical(idx * 8389, 23)
            plsc.addupdate_scatter(deg_v, [ib, idx - ib * BLK], vals)
        return _

    lax.fori_loop(0, EPW // 80, body, None)
    for ib in range(NB):
        pltpu.sync_copy(deg_v.at[ib], out_hbm.at[ib, wid])


@functools.partial(
    pl.kernel,
    out_type=jax.ShapeDtypeStruct((NC, N, D), jnp.float32),
    mesh=_MESH,
    scratch_types=[
        pltpu.VMEM((CH, K), jnp.int32),
        pltpu.VMEM((CH, K), jnp.int32),
        [pltpu.VMEM((K, D), jnp.float32) for _ in range(PIPE)],
        pltpu.VMEM((ZR, D), jnp.float32),
        pltpu.VMEM_SHARED((N, D), jnp.float32),
        [pltpu.SemaphoreType.DMA for _ in range(PIPE)],
    ],
    compiler_params=pltpu.CompilerParams(
        needs_layout_passes=False, use_tc_tiling_on_sc=False),
)
def _sc_propagate(g_hbm, e4_hbm, out_hbm,
                  idxr_v, idxc_v, rows, zrow_v, acc_sh, sems):
    c = lax.axis_index("c")
    s = lax.axis_index("s")

    zeros16 = jnp.zeros((16,), jnp.float32)

    def zero_body(i, _):
        for k in range(D // 16):
            zrow_v[i, pl.ds(k * 16, 16)] = zeros16
        return _

    lax.fori_loop(0, ZR, zero_body, None)

    pltpu.sync_copy(e4_hbm.at[0, c, s], idxr_v)
    pltpu.sync_copy(e4_hbm.at[1, c, s], idxc_v)

    def zfill(r, _):
        pltpu.sync_copy(zrow_v, acc_sh.at[pl.ds(s * RPT + r * ZR, ZR), :])
        return _

    lax.fori_loop(0, RPT // ZR, zfill, None)
    for u in range(PIPE):
        pltpu.async_copy(g_hbm.at[idxr_v.at[u]], rows[u], sems[u])
    plsc.subcore_barrier()

    def body(t, _):
        for u in range(PIPE):
            j = PIPE * t + u
            pltpu.make_async_copy(g_hbm.at[idxr_v.at[j]], rows[u],
                                  sems[u]).wait()
            pltpu.sync_copy(rows[u], acc_sh.at[idxc_v.at[j]], add=True)
            pltpu.async_copy(g_hbm.at[idxr_v.at[j + PIPE]], rows[u], sems[u])
        return _

    lax.fori_loop(0, CH // PIPE - 1, body, None)
    for u in range(PIPE):
        j = CH - PIPE + u
        pltpu.make_async_copy(g_hbm.at[idxr_v.at[j]], rows[u], sems[u]).wait()
        pltpu.sync_copy(rows[u], acc_sh.at[idxc_v.at[j]], add=True)
    plsc.subcore_barrier()
    pltpu.sync_copy(acc_sh.at[pl.ds(s * RPT, RPT), :],
                    out_hbm.at[c, pl.ds(s * RPT, RPT), :])


def _dinv_block(deg_ref):
    deg = jnp.sum(deg_ref[0], axis=0)
    return jnp.where(deg > 0, lax.rsqrt(jnp.where(deg > 0, deg, 1.0)), 0.0)


def _tc_lin1_body(x_ref, w_ref, deg_ref, g_ref):
    h = lax.dot_general(x_ref[...], w_ref[...], (((1,), (1,)), ((), ())),
                        preferred_element_type=jnp.float32)
    g_ref[...] = h * _dinv_block(deg_ref)[:, None]


def _tc_lin1(x, w1, deg32):
    return pl.pallas_call(
        _tc_lin1_body,
        grid=(N // BLK,),
        in_specs=[
            pl.BlockSpec((BLK, D), lambda i: (i, 0)),
            pl.BlockSpec((D, D), lambda i: (0, 0)),
            pl.BlockSpec((1, NW, BLK), lambda i: (i, 0, 0)),
        ],
        out_specs=pl.BlockSpec((BLK, D), lambda i: (i, 0)),
        out_shape=jax.ShapeDtypeStruct((N, D), jnp.float32),
    )(x, w1, deg32)


def _tc_mid_body(p_ref, deg_ref, b1_ref, w2_ref, g_ref):
    dinv = _dinv_block(deg_ref)[:, None]
    t = dinv * (p_ref[0] + p_ref[1]) + b1_ref[...]
    t = jnp.where(t >= 0, t, 0.01 * t)
    h = lax.dot_general(t, w2_ref[...], (((1,), (1,)), ((), ())),
                        preferred_element_type=jnp.float32)
    g_ref[...] = h * dinv


def _tc_mid(p, deg32, b1, w2):
    return pl.pallas_call(
        _tc_mid_body,
        grid=(N // BLK,),
        in_specs=[
            pl.BlockSpec((NC, BLK, D), lambda i: (0, i, 0)),
            pl.BlockSpec((1, NW, BLK), lambda i: (i, 0, 0)),
            pl.BlockSpec((1, D), lambda i: (0, 0)),
            pl.BlockSpec((D, D), lambda i: (0, 0)),
        ],
        out_specs=pl.BlockSpec((BLK, D), lambda i: (i, 0)),
        out_shape=jax.ShapeDtypeStruct((N, D), jnp.float32),
    )(p, deg32, b1, w2)


def _tc_final_body(q_ref, deg_ref, b2_ref, bat_ref, wm_ref, bm_ref,
                   out_ref, sums_ref, cnt_ref):
    i = pl.program_id(0)

    @pl.when(i == 0)
    def _():
        sums_ref[...] = jnp.zeros_like(sums_ref)
        cnt_ref[...] = jnp.zeros_like(cnt_ref)

    t = _dinv_block(deg_ref)[:, None] * (q_ref[0] + q_ref[1]) + b2_ref[...]
    h = jnp.where(t >= 0, t, 0.01 * t)
    onehot = (lax.broadcasted_iota(jnp.int32, (BLK, G), 1)
              == bat_ref[...]).astype(jnp.float32)
    sums_ref[...] += lax.dot_general(onehot, h, (((0,), (0,)), ((), ())),
                                     preferred_element_type=jnp.float32)
    cnt_ref[...] += jnp.sum(onehot, axis=0)[:, None]

    @pl.when(i == N // BLK - 1)
    def _():
        pooled = sums_ref[...] / jnp.maximum(cnt_ref[...], 1.0)
        out_ref[...] = lax.dot_general(
            pooled, wm_ref[...], (((1,), (1,)), ((), ())),
            preferred_element_type=jnp.float32) + bm_ref[...]


def _tc_final(q, deg32, b2, bat2d, wm, bm):
    return pl.pallas_call(
        _tc_final_body,
        grid=(N // BLK,),
        in_specs=[
            pl.BlockSpec((NC, BLK, D), lambda i: (0, i, 0)),
            pl.BlockSpec((1, NW, BLK), lambda i: (i, 0, 0)),
            pl.BlockSpec((1, D), lambda i: (0, 0)),
            pl.BlockSpec((BLK, 1), lambda i: (i, 0)),
            pl.BlockSpec((2, D), lambda i: (0, 0)),
            pl.BlockSpec((1, 2), lambda i: (0, 0)),
        ],
        out_specs=pl.BlockSpec((G, 2), lambda i: (0, 0)),
        out_shape=jax.ShapeDtypeStruct((G, 2), jnp.float32),
        scratch_shapes=[
            pltpu.VMEM((G, D), jnp.float32),
            pltpu.VMEM((G, 1), jnp.float32),
        ],
    )(q, deg32, b2, bat2d, wm, bm)


def kernel(x, edge_index, edge_weights, batch, W1, b1, W2, b2, Wm, bm):
    ei = edge_index.astype(jnp.int32)
    e2 = ei.reshape(2, NW, EPW)
    e4 = ei.reshape(2, NC, NS, CH, K)
    w2 = edge_weights.reshape(NW, EPW)

    deg32 = _sc_degree(e2, w2, jnp.zeros((NB, BLK), jnp.float32))
    g0 = _tc_lin1(x, W1, deg32)
    p = _sc_propagate(g0, e4)
    g1 = _tc_mid(p, deg32, b1.reshape(1, D), W2)
    q = _sc_propagate(g1, e4)
    return _tc_final(q, deg32, b2.reshape(1, D),
                     batch.astype(jnp.int32).reshape(N, 1), Wm,
                     bm.reshape(1, 2))
